# Initial kernel scaffold; baseline (speedup 1.0000x reference)
import jax
import jax.numpy as jnp
from jax.experimental import pallas as pl


def kernel(x, emb, bias):
    raise NotImplementedError("write your pallas kernel here")


# speedup vs baseline: 1.4034x; 1.4034x over previous
import functools

import jax
import jax.numpy as jnp
from jax import lax
from jax.experimental import pallas as pl
from jax.experimental.pallas import tpu as pltpu, tpu_sc as plsc

NUM_FIELDS = 26
FIELD_DIM = 40000
BATCH = 16384
L = 16


def _make_kernel(nw):
    b_per_w = BATCH // nw
    n_idx = b_per_w * NUM_FIELDS
    n_chunks = b_per_w // L
    g_chunk = 128
    n_gath = n_idx // g_chunk

    mesh = plsc.VectorSubcoreMesh(core_axis_name="c", subcore_axis_name="s")

    @functools.partial(
        pl.kernel,
        mesh=mesh,
        out_type=jax.ShapeDtypeStruct((BATCH,), jnp.float32),
        scratch_types=[
            pltpu.VMEM((n_idx,), jnp.int32),
            pltpu.VMEM((n_idx,), jnp.int32),
            pltpu.VMEM((n_idx,), jnp.float32),
            pltpu.VMEM((b_per_w,), jnp.float32),
            pltpu.VMEM((L,), jnp.float32),
            pltpu.SemaphoreType.DMA,
            pltpu.SemaphoreType.DMA,
        ],
    )
    def k(xt_hbm, emb_hbm, bias_hbm, out_hbm, xv, idxv, rowsv, outv, biasv,
          sem, xsem):
        ncores = lax.axis_size("c")
        wid = lax.axis_index("s") * ncores + lax.axis_index("c")
        base = wid * b_per_w

        pltpu.sync_copy(bias_hbm, biasv)

        def xfire(f, _):
            pltpu.async_copy(
                xt_hbm.at[pl.ds(f * BATCH + base, b_per_w)],
                xv.at[pl.ds(f * b_per_w, b_per_w)],
                xsem,
            )
            return 0

        lax.fori_loop(0, NUM_FIELDS, xfire, 0)

        def xdrain(f, _):
            pltpu.make_async_copy(
                xt_hbm.at[pl.ds(f * BATCH + base, b_per_w)],
                xv.at[pl.ds(f * b_per_w, b_per_w)],
                xsem,
            ).wait()
            return 0

        lax.fori_loop(0, NUM_FIELDS, xdrain, 0)

        def build_f(f, _):
            def build_c(c, _):
                o = f * b_per_w + c * L
                idxv[pl.ds(o, L)] = xv[pl.ds(o, L)] + f * FIELD_DIM
                return 0
            return lax.fori_loop(0, n_chunks, build_c, 0)

        lax.fori_loop(0, NUM_FIELDS, build_f, 0)

        def fire(j, _):
            pltpu.async_copy(
                emb_hbm.at[idxv.at[pl.ds(j * g_chunk, g_chunk)]],
                rowsv.at[pl.ds(j * g_chunk, g_chunk)],
                sem,
            )
            return 0

        lax.fori_loop(0, n_gath, fire, 0)

        def drain(j, _):
            pltpu.make_async_copy(
                emb_hbm.at[idxv.at[pl.ds(j * g_chunk, g_chunk)]],
                rowsv.at[pl.ds(j * g_chunk, g_chunk)],
                sem,
            ).wait()
            return 0

        lax.fori_loop(0, n_gath, drain, 0)

        bias_vec = biasv[...]

        def red_c(c, _):
            def red_f(f, acc):
                return acc + rowsv[pl.ds(f * b_per_w + c * L, L)]

            outv[pl.ds(c * L, L)] = lax.fori_loop(0, NUM_FIELDS, red_f, bias_vec)
            return 0

        lax.fori_loop(0, n_chunks, red_c, 0)

        pltpu.sync_copy(outv, out_hbm.at[pl.ds(base, b_per_w)])

    return k


def kernel(x, emb, bias):
    info = plsc.get_sparse_core_info()
    nw = info.num_cores * info.num_subcores
    xt_flat = jnp.transpose(x).reshape(-1)
    emb_flat = emb.reshape(-1)
    bias16 = jnp.broadcast_to(bias, (L,))
    out = _make_kernel(nw)(xt_flat, emb_flat, bias16)
    return out.reshape(BATCH, 1)


# speedup vs baseline: 2.6829x; 1.9117x over previous
import functools

import jax
import jax.numpy as jnp
from jax import lax
from jax.experimental import pallas as pl
from jax.experimental.pallas import tpu as pltpu, tpu_sc as plsc

NUM_FIELDS = 26
FIELD_DIM = 40000
BATCH = 16384
L = 16


def _make_kernel(nw):
    b_per_w = BATCH // nw
    n_idx = b_per_w * NUM_FIELDS
    n_chunks = b_per_w // L
    g_chunk = 128
    n_gath = n_idx // g_chunk

    mesh = plsc.VectorSubcoreMesh(core_axis_name="c", subcore_axis_name="s")

    @functools.partial(
        pl.kernel,
        mesh=mesh,
        out_type=jax.ShapeDtypeStruct((BATCH,), jnp.float32),
        scratch_types=[
            pltpu.VMEM((n_idx,), jnp.int32),
            pltpu.VMEM((n_idx,), jnp.int32),
            pltpu.VMEM((n_idx,), jnp.float32),
            pltpu.VMEM((b_per_w,), jnp.float32),
            pltpu.VMEM((L,), jnp.float32),
            pltpu.SemaphoreType.DMA,
            pltpu.SemaphoreType.DMA,
        ],
    )
    def k(xt_hbm, emb2d_hbm, bias_hbm, out_hbm, xv, idxv, rowsv, outv, biasv,
          sem, xsem):
        emb_hbm = emb2d_hbm.at[0]
        ncores = lax.axis_size("c")
        wid = lax.axis_index("s") * ncores + lax.axis_index("c")
        base = wid * b_per_w

        pltpu.sync_copy(bias_hbm, biasv)

        def xfire(f, _):
            pltpu.async_copy(
                xt_hbm.at[pl.ds(f * BATCH + base, b_per_w)],
                xv.at[pl.ds(f * b_per_w, b_per_w)],
                xsem,
            )
            return 0

        lax.fori_loop(0, NUM_FIELDS, xfire, 0)

        def xdrain(f, _):
            pltpu.make_async_copy(
                xt_hbm.at[pl.ds(f * BATCH + base, b_per_w)],
                xv.at[pl.ds(f * b_per_w, b_per_w)],
                xsem,
            ).wait()
            return 0

        lax.fori_loop(0, NUM_FIELDS, xdrain, 0)

        def build_f(f, _):
            def build_c(c, _):
                o = f * b_per_w + c * L
                idxv[pl.ds(o, L)] = xv[pl.ds(o, L)] + f * FIELD_DIM
                return 0
            return lax.fori_loop(0, n_chunks, build_c, 0)

        lax.fori_loop(0, NUM_FIELDS, build_f, 0)

        def fire(j, _):
            pltpu.async_copy(
                emb_hbm.at[idxv.at[pl.ds(j * g_chunk, g_chunk)]],
                rowsv.at[pl.ds(j * g_chunk, g_chunk)],
                sem,
            )
            return 0

        lax.fori_loop(0, n_gath, fire, 0)

        def drain(j, _):
            pltpu.make_async_copy(
                emb_hbm.at[idxv.at[pl.ds(j * g_chunk, g_chunk)]],
                rowsv.at[pl.ds(j * g_chunk, g_chunk)],
                sem,
            ).wait()
            return 0

        lax.fori_loop(0, n_gath, drain, 0)

        bias_vec = biasv[...]

        def red_c(c, _):
            def red_f(f, acc):
                return acc + rowsv[pl.ds(f * b_per_w + c * L, L)]

            outv[pl.ds(c * L, L)] = lax.fori_loop(0, NUM_FIELDS, red_f, bias_vec)
            return 0

        lax.fori_loop(0, n_chunks, red_c, 0)

        pltpu.sync_copy(outv, out_hbm.at[pl.ds(base, b_per_w)])

    return k


def kernel(x, emb, bias):
    info = plsc.get_sparse_core_info()
    nw = info.num_cores * info.num_subcores
    xt_flat = jnp.transpose(x).reshape(-1)
    bias16 = jnp.broadcast_to(bias, (L,))
    out = _make_kernel(nw)(xt_flat, emb.reshape(1, -1), bias16)
    return out.reshape(BATCH, 1)


# speedup vs baseline: 2.7725x; 1.0334x over previous
import functools

import jax
import jax.numpy as jnp
from jax import lax
from jax.experimental import pallas as pl
from jax.experimental.pallas import tpu as pltpu, tpu_sc as plsc

NUM_FIELDS = 26
FIELD_DIM = 40000
BATCH = 16384
L = 16


def _make_kernel(nw):
    b_per_w = BATCH // nw
    n_idx = b_per_w * NUM_FIELDS
    n_chunks = b_per_w // L
    g_chunk = 128
    gpf = b_per_w // g_chunk

    mesh = plsc.VectorSubcoreMesh(core_axis_name="c", subcore_axis_name="s")

    @functools.partial(
        pl.kernel,
        mesh=mesh,
        out_type=jax.ShapeDtypeStruct((BATCH,), jnp.float32),
        scratch_types=[
            pltpu.VMEM((n_idx,), jnp.int32),
            pltpu.VMEM((n_idx,), jnp.int32),
            pltpu.VMEM((n_idx,), jnp.float32),
            pltpu.VMEM((b_per_w,), jnp.float32),
            pltpu.VMEM((L,), jnp.float32),
            pltpu.SemaphoreType.DMA,
            pltpu.SemaphoreType.DMA,
        ],
    )
    def k(xt_hbm, emb2d_hbm, bias_hbm, out_hbm, xv, idxv, rowsv, outv, biasv,
          sem, xsem):
        emb_hbm = emb2d_hbm.at[0]
        ncores = lax.axis_size("c")
        wid = lax.axis_index("s") * ncores + lax.axis_index("c")
        base = wid * b_per_w

        for f in range(NUM_FIELDS):
            pltpu.async_copy(
                xt_hbm.at[pl.ds(f * BATCH + base, b_per_w)],
                xv.at[pl.ds(f * b_per_w, b_per_w)],
                xsem,
            )
        pltpu.sync_copy(bias_hbm, biasv)
        for f in range(NUM_FIELDS):
            pltpu.make_async_copy(
                xt_hbm.at[pl.ds(f * BATCH + base, b_per_w)],
                xv.at[pl.ds(f * b_per_w, b_per_w)],
                xsem,
            ).wait()

        def build_c(c, _):
            for f in range(NUM_FIELDS):
                o = f * b_per_w + c * L
                idxv[pl.ds(o, L)] = xv[pl.ds(o, L)] + f * FIELD_DIM
            return 0

        lax.fori_loop(0, n_chunks, build_c, 0, unroll=4)

        def fire(j, _):
            for q in range(gpf):
                o = j * b_per_w + q * g_chunk
                pltpu.async_copy(
                    emb_hbm.at[idxv.at[pl.ds(o, g_chunk)]],
                    rowsv.at[pl.ds(o, g_chunk)],
                    sem,
                )
            return 0

        lax.fori_loop(0, NUM_FIELDS, fire, 0)

        def drain(j, _):
            for q in range(gpf):
                o = j * b_per_w + q * g_chunk
                pltpu.make_async_copy(
                    emb_hbm.at[idxv.at[pl.ds(o, g_chunk)]],
                    rowsv.at[pl.ds(o, g_chunk)],
                    sem,
                ).wait()
            return 0

        lax.fori_loop(0, NUM_FIELDS, drain, 0)

        bias_vec = biasv[...]

        def red_c(c, _):
            acc = bias_vec
            for f in range(NUM_FIELDS):
                acc = acc + rowsv[pl.ds(f * b_per_w + c * L, L)]
            outv[pl.ds(c * L, L)] = acc
            return 0

        lax.fori_loop(0, n_chunks, red_c, 0, unroll=2)

        pltpu.sync_copy(outv, out_hbm.at[pl.ds(base, b_per_w)])

    return k


def kernel(x, emb, bias):
    info = plsc.get_sparse_core_info()
    nw = info.num_cores * info.num_subcores
    xt_flat = jnp.transpose(x).reshape(-1)
    bias16 = jnp.broadcast_to(bias, (L,))
    out = _make_kernel(nw)(xt_flat, emb.reshape(1, -1), bias16)
    return out.reshape(BATCH, 1)


# speedup vs baseline: 2.8320x; 1.0214x over previous
import functools

import jax
import jax.numpy as jnp
from jax import lax
from jax.experimental import pallas as pl
from jax.experimental.pallas import tpu as pltpu, tpu_sc as plsc

NUM_FIELDS = 26
FIELD_DIM = 40000
BATCH = 16384
L = 16


def _make_kernel(nw):
    b_per_w = BATCH // nw
    n_idx = b_per_w * NUM_FIELDS
    n_chunks = b_per_w // L
    g_chunk = 128
    gpf = b_per_w // g_chunk

    mesh = plsc.VectorSubcoreMesh(core_axis_name="c", subcore_axis_name="s")

    @functools.partial(
        pl.kernel,
        mesh=mesh,
        out_type=jax.ShapeDtypeStruct((BATCH,), jnp.float32),
        scratch_types=[
            pltpu.VMEM((n_idx,), jnp.int32),
            pltpu.VMEM((n_idx,), jnp.int32),
            pltpu.VMEM((n_idx,), jnp.float32),
            pltpu.VMEM((b_per_w,), jnp.float32),
            pltpu.VMEM((L,), jnp.float32),
            pltpu.SemaphoreType.DMA,
            pltpu.SemaphoreType.DMA,
        ],
    )
    def k(xt_hbm, emb2d_hbm, bias_hbm, out_hbm, xv, idxv, rowsv, outv, biasv,
          sem, xsem):
        emb_hbm = emb2d_hbm.at[0]
        ncores = lax.axis_size("c")
        wid = lax.axis_index("s") * ncores + lax.axis_index("c")
        base = wid * b_per_w

        for f in range(NUM_FIELDS):
            pltpu.async_copy(
                xt_hbm.at[pl.ds(f * BATCH + base, b_per_w)],
                xv.at[pl.ds(f * b_per_w, b_per_w)],
                xsem,
            )
        pltpu.sync_copy(bias_hbm, biasv)
        for f in range(NUM_FIELDS):
            pltpu.make_async_copy(
                xt_hbm.at[pl.ds(f * BATCH + base, b_per_w)],
                xv.at[pl.ds(f * b_per_w, b_per_w)],
                xsem,
            ).wait()

        def build_fire(f, _):
            fo = f * b_per_w

            def build_c(c, _):
                o = fo + c * L
                idxv[pl.ds(o, L)] = xv[pl.ds(o, L)] + f * FIELD_DIM
                return 0

            lax.fori_loop(0, n_chunks, build_c, 0, unroll=8)
            for q in range(gpf):
                o = fo + q * g_chunk
                pltpu.async_copy(
                    emb_hbm.at[idxv.at[pl.ds(o, g_chunk)]],
                    rowsv.at[pl.ds(o, g_chunk)],
                    sem,
                )
            return 0

        lax.fori_loop(0, NUM_FIELDS, build_fire, 0)

        bias_vec = biasv[...]

        def seed_c(c, _):
            outv[pl.ds(c * L, L)] = bias_vec
            return 0

        lax.fori_loop(0, n_chunks, seed_c, 0, unroll=8)

        def drain_red(f, _):
            fo = f * b_per_w
            for q in range(gpf):
                o = fo + q * g_chunk
                pltpu.make_async_copy(
                    emb_hbm.at[idxv.at[pl.ds(o, g_chunk)]],
                    rowsv.at[pl.ds(o, g_chunk)],
                    sem,
                ).wait()

            def red_c(c, _):
                co = c * L
                outv[pl.ds(co, L)] = outv[pl.ds(co, L)] + rowsv[pl.ds(fo + co, L)]
                return 0

            lax.fori_loop(0, n_chunks, red_c, 0, unroll=8)
            return 0

        lax.fori_loop(0, NUM_FIELDS, drain_red, 0)

        pltpu.sync_copy(outv, out_hbm.at[pl.ds(base, b_per_w)])

    return k


def kernel(x, emb, bias):
    info = plsc.get_sparse_core_info()
    nw = info.num_cores * info.num_subcores
    xt_flat = jnp.transpose(x).reshape(-1)
    bias16 = jnp.broadcast_to(bias, (L,))
    out = _make_kernel(nw)(xt_flat, emb.reshape(1, -1), bias16)
    return out.reshape(BATCH, 1)


# speedup vs baseline: 2.8383x; 1.0022x over previous
import functools

import jax
import jax.numpy as jnp
from jax import lax
from jax.experimental import pallas as pl
from jax.experimental.pallas import tpu as pltpu, tpu_sc as plsc

NUM_FIELDS = 26
FIELD_DIM = 40000
BATCH = 16384
L = 16


def _make_kernel(nw):
    b_per_w = BATCH // nw
    n_idx = b_per_w * NUM_FIELDS
    n_chunks = b_per_w // L

    mesh = plsc.VectorSubcoreMesh(core_axis_name="c", subcore_axis_name="s")

    @functools.partial(
        pl.kernel,
        mesh=mesh,
        out_type=jax.ShapeDtypeStruct((BATCH,), jnp.float32),
        scratch_types=[
            pltpu.VMEM((n_idx,), jnp.int32),
            pltpu.VMEM((n_idx,), jnp.int32),
            pltpu.VMEM((n_idx,), jnp.float32),
            pltpu.VMEM((b_per_w,), jnp.float32),
            pltpu.VMEM((L,), jnp.float32),
            pltpu.SemaphoreType.DMA,
            pltpu.SemaphoreType.DMA,
        ],
    )
    def k(xt_hbm, emb2d_hbm, bias_hbm, out_hbm, xv, idxv, rowsv, outv, biasv,
          sem, xsem):
        emb_hbm = emb2d_hbm.at[0]
        ncores = lax.axis_size("c")
        wid = lax.axis_index("s") * ncores + lax.axis_index("c")
        base = wid * b_per_w

        for f in range(NUM_FIELDS):
            pltpu.async_copy(
                xt_hbm.at[pl.ds(f * BATCH + base, b_per_w)],
                xv.at[pl.ds(f * b_per_w, b_per_w)],
                xsem,
            )
        pltpu.sync_copy(bias_hbm, biasv)
        for f in range(NUM_FIELDS):
            pltpu.make_async_copy(
                xt_hbm.at[pl.ds(f * BATCH + base, b_per_w)],
                xv.at[pl.ds(f * b_per_w, b_per_w)],
                xsem,
            ).wait()

        def build_fire(f, _):
            fo = f * b_per_w

            def build_c(c, _):
                o = fo + c * L
                idxv[pl.ds(o, L)] = xv[pl.ds(o, L)] + f * FIELD_DIM
                return 0

            lax.fori_loop(0, n_chunks, build_c, 0, unroll=8)
            pltpu.async_copy(
                emb_hbm.at[idxv.at[pl.ds(fo, b_per_w)]],
                rowsv.at[pl.ds(fo, b_per_w)],
                sem,
            )
            return 0

        lax.fori_loop(0, NUM_FIELDS, build_fire, 0)

        bias_vec = biasv[...]

        def seed_c(c, _):
            outv[pl.ds(c * L, L)] = bias_vec
            return 0

        lax.fori_loop(0, n_chunks, seed_c, 0, unroll=8)

        def drain_red(f, _):
            fo = f * b_per_w
            pltpu.make_async_copy(
                emb_hbm.at[idxv.at[pl.ds(fo, b_per_w)]],
                rowsv.at[pl.ds(fo, b_per_w)],
                sem,
            ).wait()

            def red_c(c, _):
                co = c * L
                outv[pl.ds(co, L)] = outv[pl.ds(co, L)] + rowsv[pl.ds(fo + co, L)]
                return 0

            lax.fori_loop(0, n_chunks, red_c, 0, unroll=8)
            return 0

        lax.fori_loop(0, NUM_FIELDS, drain_red, 0)

        pltpu.sync_copy(outv, out_hbm.at[pl.ds(base, b_per_w)])

    return k


def kernel(x, emb, bias):
    info = plsc.get_sparse_core_info()
    nw = info.num_cores * info.num_subcores
    xt_flat = jnp.transpose(x).reshape(-1)
    bias16 = jnp.broadcast_to(bias, (L,))
    out = _make_kernel(nw)(xt_flat, emb.reshape(1, -1), bias16)
    return out.reshape(BATCH, 1)


# speedup vs baseline: 2.8457x; 1.0026x over previous
import functools

import jax
import jax.numpy as jnp
from jax import lax
from jax.experimental import pallas as pl
from jax.experimental.pallas import tpu as pltpu, tpu_sc as plsc

NUM_FIELDS = 26
FIELD_DIM = 40000
BATCH = 16384
L = 16


def _make_kernel(nw):
    b_per_w = BATCH // nw
    n_idx = b_per_w * NUM_FIELDS
    n_chunks = b_per_w // L

    mesh = plsc.VectorSubcoreMesh(core_axis_name="c", subcore_axis_name="s")

    @functools.partial(
        pl.kernel,
        mesh=mesh,
        out_type=jax.ShapeDtypeStruct((1, BATCH), jnp.float32),
        scratch_types=[
            pltpu.VMEM((n_idx,), jnp.int32),
            pltpu.VMEM((n_idx,), jnp.int32),
            pltpu.VMEM((n_idx,), jnp.float32),
            pltpu.VMEM((b_per_w,), jnp.float32),
            pltpu.VMEM((L,), jnp.float32),
            pltpu.SemaphoreType.DMA,
            pltpu.SemaphoreType.DMA,
        ],
    )
    def k(xt_hbm, emb2d_hbm, bias_hbm, out2d_hbm, xv, idxv, rowsv, outv, biasv,
          sem, xsem):
        emb_hbm = emb2d_hbm.at[0]
        out_hbm = out2d_hbm.at[0]
        ncores = lax.axis_size("c")
        wid = lax.axis_index("s") * ncores + lax.axis_index("c")
        base = wid * b_per_w

        for f in range(NUM_FIELDS):
            pltpu.async_copy(
                xt_hbm.at[pl.ds(f * BATCH + base, b_per_w)],
                xv.at[pl.ds(f * b_per_w, b_per_w)],
                xsem,
            )
        pltpu.sync_copy(bias_hbm, biasv)
        for f in range(NUM_FIELDS):
            pltpu.make_async_copy(
                xt_hbm.at[pl.ds(f * BATCH + base, b_per_w)],
                xv.at[pl.ds(f * b_per_w, b_per_w)],
                xsem,
            ).wait()

        def build_fire(f, _):
            fo = f * b_per_w

            def build_c(c, _):
                o = fo + c * L
                idxv[pl.ds(o, L)] = xv[pl.ds(o, L)] + f * FIELD_DIM
                return 0

            lax.fori_loop(0, n_chunks, build_c, 0, unroll=8)
            pltpu.async_copy(
                emb_hbm.at[idxv.at[pl.ds(fo, b_per_w)]],
                rowsv.at[pl.ds(fo, b_per_w)],
                sem,
            )
            return 0

        lax.fori_loop(0, NUM_FIELDS, build_fire, 0)

        bias_vec = biasv[...]

        def seed_c(c, _):
            outv[pl.ds(c * L, L)] = bias_vec
            return 0

        lax.fori_loop(0, n_chunks, seed_c, 0, unroll=8)

        def drain_red(f, _):
            fo = f * b_per_w
            pltpu.make_async_copy(
                emb_hbm.at[idxv.at[pl.ds(fo, b_per_w)]],
                rowsv.at[pl.ds(fo, b_per_w)],
                sem,
            ).wait()

            def red_c(c, _):
                co = c * L
                outv[pl.ds(co, L)] = outv[pl.ds(co, L)] + rowsv[pl.ds(fo + co, L)]
                return 0

            lax.fori_loop(0, n_chunks, red_c, 0, unroll=8)
            return 0

        lax.fori_loop(0, NUM_FIELDS, drain_red, 0)

        pltpu.sync_copy(outv, out_hbm.at[pl.ds(base, b_per_w)])

    return k


def kernel(x, emb, bias):
    info = plsc.get_sparse_core_info()
    nw = info.num_cores * info.num_subcores
    xt_flat = jnp.transpose(x).reshape(-1)
    bias16 = jnp.broadcast_to(bias, (L,))
    out = _make_kernel(nw)(xt_flat, emb.reshape(1, -1), bias16)
    return out.reshape(BATCH, 1)


# speedup vs baseline: 2.9076x; 1.0217x over previous
import functools

import jax
import jax.numpy as jnp
from jax import lax
from jax.experimental import pallas as pl
from jax.experimental.pallas import tpu as pltpu, tpu_sc as plsc

NUM_FIELDS = 26
FIELD_DIM = 40000
BATCH = 16384
L = 16


def _make_kernel(nw):
    b_per_w = BATCH // nw
    n_idx = b_per_w * NUM_FIELDS
    n_chunks = b_per_w // L

    mesh = plsc.VectorSubcoreMesh(core_axis_name="c", subcore_axis_name="s")

    @functools.partial(
        pl.kernel,
        mesh=mesh,
        out_type=jax.ShapeDtypeStruct((1, BATCH), jnp.float32),
        scratch_types=[
            pltpu.VMEM((n_idx,), jnp.int32),
            pltpu.VMEM((n_idx,), jnp.int32),
            pltpu.VMEM((n_idx,), jnp.float32),
            pltpu.VMEM((b_per_w,), jnp.float32),
            pltpu.VMEM((L,), jnp.float32),
            pltpu.SemaphoreType.DMA,
            pltpu.SemaphoreType.DMA,
        ],
    )
    def k(xt_hbm, emb2d_hbm, bias_hbm, out2d_hbm, xv, idxv, rowsv, outv, biasv,
          sem, xsem):
        emb_hbm = emb2d_hbm.at[0]
        out_hbm = out2d_hbm.at[0]
        ncores = lax.axis_size("c")
        wid = lax.axis_index("s") * ncores + lax.axis_index("c")
        base = wid * b_per_w

        for f in range(NUM_FIELDS):
            pltpu.async_copy(
                xt_hbm.at[pl.ds(f * BATCH + base, b_per_w)],
                xv.at[pl.ds(f * b_per_w, b_per_w)],
                xsem,
            )

        def build_fire(f, _):
            fo = f * b_per_w
            pltpu.make_async_copy(
                xt_hbm.at[pl.ds(f * BATCH + base, b_per_w)],
                xv.at[pl.ds(fo, b_per_w)],
                xsem,
            ).wait()

            def build_c(c, _):
                o = fo + c * L
                idxv[pl.ds(o, L)] = xv[pl.ds(o, L)] + f * FIELD_DIM
                return 0

            lax.fori_loop(0, n_chunks, build_c, 0, unroll=8)
            pltpu.async_copy(
                emb_hbm.at[idxv.at[pl.ds(fo, b_per_w)]],
                rowsv.at[pl.ds(fo, b_per_w)],
                sem,
            )
            return 0

        lax.fori_loop(0, NUM_FIELDS, build_fire, 0)
        pltpu.sync_copy(bias_hbm, biasv)

        bias_vec = biasv[...]

        def seed_c(c, _):
            outv[pl.ds(c * L, L)] = bias_vec
            return 0

        lax.fori_loop(0, n_chunks, seed_c, 0, unroll=8)

        def drain_red(f, _):
            fo = f * b_per_w
            pltpu.make_async_copy(
                emb_hbm.at[idxv.at[pl.ds(fo, b_per_w)]],
                rowsv.at[pl.ds(fo, b_per_w)],
                sem,
            ).wait()

            def red_c(c, _):
                co = c * L
                outv[pl.ds(co, L)] = outv[pl.ds(co, L)] + rowsv[pl.ds(fo + co, L)]
                return 0

            lax.fori_loop(0, n_chunks, red_c, 0, unroll=8)
            return 0

        lax.fori_loop(0, NUM_FIELDS, drain_red, 0)

        pltpu.sync_copy(outv, out_hbm.at[pl.ds(base, b_per_w)])

    return k


def kernel(x, emb, bias):
    info = plsc.get_sparse_core_info()
    nw = info.num_cores * info.num_subcores
    xt_flat = jnp.transpose(x).reshape(-1)
    bias16 = jnp.broadcast_to(bias, (L,))
    out = _make_kernel(nw)(xt_flat, emb.reshape(1, -1), bias16)
    return out.reshape(BATCH, 1)
